# baseline (speedup 1.0000x reference)
import functools

import jax
import jax.numpy as jnp
from jax import lax
from jax.experimental import pallas as pl
from jax.experimental.pallas import tpu as pltpu
from jax.experimental.pallas import tpu_sc as plsc

VOCAB = 23236
DIM = 1728
DIM_PAD = 1792
BATCH = 1024
SEQ = 50
SEQ_PAD = 56
NP = BATCH * SEQ_PAD
NC, NS = 2, 16
NW = NC * NS
ROWS_PER_W = NP // NW
CH = 32
NCHUNK = ROWS_PER_W // CH

PAD_BR = 256


def _emb_body(table_hbm, idx_hbm, out_hbm, idx_v, rows_v, gsem, ssem):
    wid = lax.axis_index("s") * NC + lax.axis_index("c")
    base = wid * ROWS_PER_W

    pltpu.sync_copy(idx_hbm.at[wid], idx_v)

    def gather(c, slot):
        return pltpu.async_copy(table_hbm.at[idx_v.at[c]], rows_v.at[slot], gsem)

    def scatter(c, slot):
        return pltpu.async_copy(
            rows_v.at[slot], out_hbm.at[pl.ds(base + c * CH, CH)], ssem)

    def wait_gather(slot):
        pltpu.make_async_copy(table_hbm.at[idx_v.at[0]], rows_v.at[slot], gsem).wait()

    def wait_scatter(c, slot):
        pltpu.make_async_copy(
            rows_v.at[slot], out_hbm.at[pl.ds(base + c * CH, CH)], ssem).wait()

    gather(0, 0)

    wait_gather(0)
    scatter(0, 0)
    gather(1, 1)

    def pair(t, _):
        c1 = 2 * t - 1
        wait_gather(1)
        scatter(c1, 1)
        wait_scatter(c1 - 1, 0)
        gather(c1 + 1, 0)
        c2 = 2 * t
        wait_gather(0)
        scatter(c2, 0)
        wait_scatter(c2 - 1, 1)
        gather(c2 + 1, 1)
        return _

    lax.fori_loop(1, NCHUNK // 2, pair, 0)

    wait_gather(1)
    scatter(NCHUNK - 1, 1)
    wait_scatter(NCHUNK - 2, 0)
    wait_scatter(NCHUNK - 1, 1)


def _pad_body(w_ref, o_ref):
    o_ref[...] = jnp.concatenate(
        [w_ref[...], jnp.zeros((PAD_BR, DIM_PAD - DIM), jnp.float32)], axis=1)


def _depad_body(i_ref, o_ref):
    o_ref[0] = i_ref[:SEQ, :DIM]


@jax.jit
def _emb(weight, idx):
    wpad = pl.pallas_call(
        _pad_body,
        grid=(pl.cdiv(VOCAB, PAD_BR),),
        in_specs=[pl.BlockSpec((PAD_BR, DIM), lambda g: (g, 0))],
        out_specs=pl.BlockSpec((PAD_BR, DIM_PAD), lambda g: (g, 0)),
        out_shape=jax.ShapeDtypeStruct((VOCAB, DIM_PAD), jnp.float32),
    )(weight)

    mesh = plsc.VectorSubcoreMesh(
        core_axis_name="c", subcore_axis_name="s", num_cores=NC, num_subcores=NS)
    f = pl.kernel(
        _emb_body,
        out_type=jax.ShapeDtypeStruct((NP, DIM_PAD), jnp.float32),
        mesh=mesh,
        scratch_types=[
            pltpu.VMEM((NCHUNK, CH), jnp.int32),
            pltpu.VMEM((2, CH, DIM_PAD), jnp.float32),
            pltpu.SemaphoreType.DMA,
            pltpu.SemaphoreType.DMA,
        ],
    )
    gathered = f(wpad, idx)

    return pl.pallas_call(
        _depad_body,
        grid=(BATCH,),
        in_specs=[pl.BlockSpec((SEQ_PAD, DIM_PAD), lambda b: (b, 0))],
        out_specs=pl.BlockSpec((1, SEQ, DIM), lambda b: (b, 0, 0)),
        out_shape=jax.ShapeDtypeStruct((BATCH, SEQ, DIM), jnp.float32),
    )(gathered)


def kernel(input_ids, weight):
    idx = jnp.pad(input_ids, ((0, 0), (0, SEQ_PAD - SEQ)))
    return _emb(weight, idx.reshape(NW, NCHUNK, CH))


# speedup vs baseline: 1.2670x; 1.2670x over previous
import functools

import jax
import jax.numpy as jnp
from jax import lax
from jax.experimental import pallas as pl
from jax.experimental.pallas import tpu as pltpu
from jax.experimental.pallas import tpu_sc as plsc

VOCAB = 23236
DIM = 1728
DIM_PAD = 1792
BATCH = 1024
SEQ = 50
SEQ_PAD = 56
NP = BATCH * SEQ_PAD
NC, NS = 2, 16
NW = NC * NS
ROWS_PER_W = NP // NW
CH = 16
NBUF = 4
NCHUNK = ROWS_PER_W // CH

PAD_BR = 256
DEPAD_NB = 8


def _emb_body(table_hbm, idx_hbm, out_hbm, idx_v, rows_v, gsem, ssem):
    wid = lax.axis_index("s") * NC + lax.axis_index("c")
    base = wid * ROWS_PER_W

    pltpu.sync_copy(idx_hbm.at[wid], idx_v)

    def gather(c, slot):
        return pltpu.async_copy(
            table_hbm.at[idx_v.at[c]], rows_v.at[slot], gsem.at[slot])

    def scatter(c, slot):
        return pltpu.async_copy(
            rows_v.at[slot], out_hbm.at[pl.ds(base + c * CH, CH)], ssem.at[slot])

    def wait_gather(slot):
        pltpu.make_async_copy(
            table_hbm.at[idx_v.at[0]], rows_v.at[slot], gsem.at[slot]).wait()

    def wait_scatter(c, slot):
        pltpu.make_async_copy(
            rows_v.at[slot], out_hbm.at[pl.ds(base + c * CH, CH)],
            ssem.at[slot]).wait()

    gather(0, 0)
    gather(1, 1)

    def step(c, slot, first, last):
        wait_gather(slot)
        scatter(c, slot)
        if not first:
            wait_scatter(c - 2, (c + 2) % NBUF)
        if not last:
            gather(c + 2, (c + 2) % NBUF)

    step(0, 0, True, False)
    step(1, 1, True, False)
    step(2, 2, False, False)
    step(3, 3, False, False)

    def quad(t, _):
        c = 4 * t
        step(c, 0, False, False)
        step(c + 1, 1, False, False)
        step(c + 2, 2, False, False)
        step(c + 3, 3, False, False)
        return _

    lax.fori_loop(1, NCHUNK // 4 - 1, quad, 0)

    step(NCHUNK - 4, 0, False, False)
    step(NCHUNK - 3, 1, False, False)
    step(NCHUNK - 2, 2, False, True)
    step(NCHUNK - 1, 3, False, True)
    wait_scatter(NCHUNK - 2, 2)
    wait_scatter(NCHUNK - 1, 3)


def _pad_body(w_ref, o_ref):
    o_ref[...] = jnp.concatenate(
        [w_ref[...], jnp.zeros((PAD_BR, DIM_PAD - DIM), jnp.float32)], axis=1)


def _depad_body(i_ref, o_ref):
    for i in range(DEPAD_NB):
        o_ref[i] = i_ref[pl.ds(i * SEQ_PAD, SEQ), :DIM]


@jax.jit
def _emb(weight, idx):
    wpad = pl.pallas_call(
        _pad_body,
        grid=(pl.cdiv(VOCAB, PAD_BR),),
        in_specs=[pl.BlockSpec((PAD_BR, DIM), lambda g: (g, 0))],
        out_specs=pl.BlockSpec((PAD_BR, DIM_PAD), lambda g: (g, 0)),
        out_shape=jax.ShapeDtypeStruct((VOCAB, DIM_PAD), jnp.float32),
    )(weight)

    mesh = plsc.VectorSubcoreMesh(
        core_axis_name="c", subcore_axis_name="s", num_cores=NC, num_subcores=NS)
    f = pl.kernel(
        _emb_body,
        out_type=jax.ShapeDtypeStruct((NP, DIM_PAD), jnp.float32),
        mesh=mesh,
        scratch_types=[
            pltpu.VMEM((NCHUNK, CH), jnp.int32),
            pltpu.VMEM((NBUF, CH, DIM_PAD), jnp.float32),
            pltpu.SemaphoreType.DMA((NBUF,)),
            pltpu.SemaphoreType.DMA((NBUF,)),
        ],
    )
    gathered = f(wpad, idx)

    return pl.pallas_call(
        _depad_body,
        grid=(BATCH // DEPAD_NB,),
        in_specs=[pl.BlockSpec((DEPAD_NB * SEQ_PAD, DIM_PAD), lambda g: (g, 0))],
        out_specs=pl.BlockSpec((DEPAD_NB, SEQ, DIM), lambda g: (g, 0, 0)),
        out_shape=jax.ShapeDtypeStruct((BATCH, SEQ, DIM), jnp.float32),
    )(gathered)


def kernel(input_ids, weight):
    idx = jnp.pad(input_ids, ((0, 0), (0, SEQ_PAD - SEQ)))
    return _emb(weight, idx.reshape(NW, NCHUNK, CH))
